# baseline (speedup 1.0000x reference)
import functools

import jax
import jax.numpy as jnp
from jax import lax
from jax.experimental import pallas as pl
from jax.experimental.pallas import tpu as pltpu
from jax.experimental.pallas import tpu_sc as plsc

VOCAB = 1000000
MAXLEN = 200
EMBED_DIM = 64
BATCH = 4096

NUM_CORES = 2
NUM_SUBCORES = 16
LANES = 16
NW = NUM_CORES * NUM_SUBCORES
BCH = BATCH // NW
DQ = EMBED_DIM // LANES
PITCH = BCH + 1
NBUF = 4
TBUF = 2
NCHUNK = MAXLEN // 2
NGROUPS = NCHUNK // NBUF
LH = MAXLEN // 8


def _make_kernel():
    mesh = plsc.VectorSubcoreMesh(core_axis_name="c", subcore_axis_name="s")

    @functools.partial(
        pl.kernel,
        out_type=jax.ShapeDtypeStruct((MAXLEN, 8, NW, 8, BCH), jnp.float32),
        name="tok_pos_embed",
        mesh=mesh,
        scratch_types=[
            pltpu.VMEM((EMBED_DIM, MAXLEN), jnp.float32),
            pltpu.VMEM((LH, 8 * BCH), jnp.int32),
            pltpu.VMEM((NBUF, 2 * BCH, EMBED_DIM), jnp.float32),
            pltpu.VMEM((TBUF, 8, 8, PITCH), jnp.float32),
            pltpu.SemaphoreType.DMA,
            pltpu.SemaphoreType.DMA,
            pltpu.SemaphoreType.DMA,
            pltpu.SemaphoreType.DMA,
            pltpu.SemaphoreType.DMA,
            pltpu.SemaphoreType.DMA,
        ],
        compiler_params=pltpu.CompilerParams(use_tc_tiling_on_sc=False,
                                             needs_layout_passes=False),
    )
    def tok_pos_embed(x_hbm, tok_hbm, pos_hbm, out_hbm,
                      pos_v, idx_v, gbuf, tbuf,
                      g0, g1, g2, g3, o0, o1):
        wid = lax.axis_index("s") * NUM_CORES + lax.axis_index("c")
        gsem = (g0, g1, g2, g3)
        osem = (o0, o1)
        pltpu.sync_copy(pos_hbm, pos_v)
        pltpu.sync_copy(x_hbm.at[:, wid, :], idx_v)

        def chunk_idx(c):
            return idx_v.at[c >> 2, pl.ds((c & 3) * 2 * BCH, 2 * BCH)]

        def start_gather(c, bb):
            pltpu.async_copy(tok_hbm.at[chunk_idx(c)], gbuf.at[bb], gsem[bb])

        def wait_gather(c, bb):
            pltpu.make_async_copy(tok_hbm.at[chunk_idx(c)],
                                  gbuf.at[bb], gsem[bb]).wait()

        for bb in range(NBUF):
            start_gather(bb, bb)

        rows_hi = [(lax.iota(jnp.int32, LANES) + dq * LANES) >> 3
                   for dq in range(DQ)]
        rows_lo = [(lax.iota(jnp.int32, LANES) + dq * LANES) & 7
                   for dq in range(DQ)]
        rows_dq = [lax.iota(jnp.int32, LANES) + dq * LANES for dq in range(DQ)]

        def wait_out(tb):
            pltpu.make_async_copy(
                tbuf.at[tb, :, :, pl.ds(0, BCH)],
                out_hbm.at[0, :, 0, :, :], osem[tb]).wait()

        def group_body(g, carry):
            for bb in range(NBUF):
                c = g * NBUF + bb
                wait_gather(c, bb)

                for half in range(2):
                    tb = half
                    l = 2 * c + half

                    if bb == 0:
                        pl.when(g >= 1)(functools.partial(wait_out, tb))
                    else:
                        wait_out(tb)

                    l_splat = jnp.full((LANES,), l, jnp.int32)
                    posc = [plsc.load_gather(pos_v, [rows_dq[dq], l_splat])
                            for dq in range(DQ)]

                    def per_token(r, cr):
                        cols = jnp.full((LANES,), r, jnp.int32)
                        for dq in range(DQ):
                            v = (gbuf[bb, half * BCH + r, pl.ds(dq * LANES, LANES)]
                                 + posc[dq])
                            plsc.store_scatter(
                                tbuf.at[tb],
                                [rows_hi[dq], rows_lo[dq], cols], v)
                        return cr

                    lax.fori_loop(0, BCH, per_token, 0, unroll=4)

                    pltpu.async_copy(
                        tbuf.at[tb, :, :, pl.ds(0, BCH)],
                        out_hbm.at[l, :, wid, :, :], osem[tb])

                @pl.when(g * NBUF + bb + NBUF < NCHUNK)
                def _next_gather():
                    start_gather(c + NBUF, bb)
            return carry

        lax.fori_loop(0, NGROUPS, group_body, 0)
        for tb in range(TBUF):
            wait_out(tb)

    return tok_pos_embed


_kernel_call = _make_kernel()


def kernel(x, token_table, pos_table):
    x4 = (x.astype(jnp.int32)
          .transpose(1, 0)
          .reshape(LH, 8, NW, BCH)
          .transpose(0, 2, 1, 3)
          .reshape(LH, NW, 8 * BCH))
    pos_t = jnp.transpose(pos_table, (1, 0))
    out5 = _kernel_call(x4, token_table, pos_t)
    return (out5.transpose(2, 4, 0, 1, 3)
            .reshape(BATCH, MAXLEN, EMBED_DIM))
